# Initial kernel scaffold; baseline (speedup 1.0000x reference)
import jax
import jax.numpy as jnp
from jax.experimental import pallas as pl


def kernel(w, scores):
    raise NotImplementedError("write your pallas kernel here")


# speedup vs baseline: 36.8171x; 36.8171x over previous
import functools

import jax
import jax.numpy as jnp
from jax import lax
from jax.experimental import pallas as pl
from jax.experimental.pallas import tpu as pltpu

_PRUNE_RATE = 0.5


def _sel_kernel(s_ref, t_ref, hist_ref, state_ref, *, nb, k, npass):
    p = pl.program_id(0)
    b = pl.program_id(1)

    @pl.when((p == 0) & (b == 0))
    def _init():
        state_ref[0] = jnp.int32(k)
        state_ref[1] = jnp.int32(0)

    @pl.when(b == 0)
    def _zero():
        hist_ref[...] = jnp.zeros_like(hist_ref)

    bits = lax.bitcast_convert_type(s_ref[...], jnp.int32)
    u = bits & jnp.int32(0x7FFFFFFF)
    shift = 28 - 4 * p
    nib = (u >> shift) & 15
    sh_hi = jnp.minimum(shift + 4, 31)
    prefix = state_ref[1]
    match = (u >> sh_hi) == (prefix >> sh_hi)
    for bin_ in range(16):
        sel = jnp.where(match & (nib == bin_), jnp.int32(1), jnp.int32(0))
        hist_ref[bin_ : bin_ + 1, :] += jnp.sum(sel, axis=0, keepdims=True)

    @pl.when(b == nb - 1)
    def _finalize():
        h = jnp.sum(hist_ref[...], axis=1)
        bi = lax.broadcasted_iota(jnp.int32, (16, 16), 0)
        ji = lax.broadcasted_iota(jnp.int32, (16, 16), 1)
        sx = jnp.sum(jnp.where(ji > bi, h[None, :], 0), axis=1)
        need = state_ref[0]
        bstar = jnp.sum(jnp.where(sx >= need, jnp.int32(1), jnp.int32(0)))
        sxb = jnp.sum(jnp.where(lax.iota(jnp.int32, 16) == bstar, sx, 0))
        state_ref[0] = need - sxb
        state_ref[1] = prefix | (bstar << shift)

    @pl.when((p == npass - 1) & (b == nb - 1))
    def _emit():
        t_ref[...] = jnp.full_like(t_ref, state_ref[1])


def _sum_kernel(t_ref, w_ref, s_ref, out_ref, acc_ref, *, nb):
    b = pl.program_id(0)

    @pl.when(b == 0)
    def _init():
        acc_ref[0] = jnp.float32(0.0)

    bits = lax.bitcast_convert_type(s_ref[...], jnp.int32)
    u = bits & jnp.int32(0x7FFFFFFF)
    t = t_ref[0, 0]
    acc_ref[0] += jnp.sum(jnp.where(u >= t, jnp.abs(w_ref[...]), 0.0))

    @pl.when(b == nb - 1)
    def _emit():
        out_ref[...] = jnp.full_like(out_ref, acc_ref[0])


def _out_kernel(t_ref, ssum_ref, w_ref, s_ref, o_ref, *, inv_k):
    bits = lax.bitcast_convert_type(s_ref[...], jnp.int32)
    u = bits & jnp.int32(0x7FFFFFFF)
    t = t_ref[0, 0]
    alpha = ssum_ref[0, 0] * inv_k
    w = w_ref[...]
    o_ref[...] = jnp.where(u >= t, alpha * jnp.sign(w), jnp.float32(0.0))


def kernel(w, scores):
    rows, cols = scores.shape
    n = rows * cols
    num_unpruned = int(_PRUNE_RATE * n)
    topk = n - int((1.0 - _PRUNE_RATE) * n)
    br = min(256, rows)
    nb = rows // br
    npass = 8

    t_arr = pl.pallas_call(
        functools.partial(_sel_kernel, nb=nb, k=topk, npass=npass),
        grid=(npass, nb),
        in_specs=[pl.BlockSpec((br, cols), lambda p, b: (b, 0))],
        out_specs=pl.BlockSpec((8, 128), lambda p, b: (0, 0)),
        out_shape=jax.ShapeDtypeStruct((8, 128), jnp.int32),
        scratch_shapes=[
            pltpu.VMEM((16, cols), jnp.int32),
            pltpu.SMEM((2,), jnp.int32),
        ],
    )(scores)

    ssum = pl.pallas_call(
        functools.partial(_sum_kernel, nb=nb),
        grid=(nb,),
        in_specs=[
            pl.BlockSpec((8, 128), lambda b: (0, 0)),
            pl.BlockSpec((br, cols), lambda b: (b, 0)),
            pl.BlockSpec((br, cols), lambda b: (b, 0)),
        ],
        out_specs=pl.BlockSpec((8, 128), lambda b: (0, 0)),
        out_shape=jax.ShapeDtypeStruct((8, 128), jnp.float32),
        scratch_shapes=[pltpu.SMEM((1,), jnp.float32)],
    )(t_arr, w, scores)

    out = pl.pallas_call(
        functools.partial(_out_kernel, inv_k=1.0 / num_unpruned),
        grid=(nb,),
        in_specs=[
            pl.BlockSpec((8, 128), lambda b: (0, 0)),
            pl.BlockSpec((8, 128), lambda b: (0, 0)),
            pl.BlockSpec((br, cols), lambda b: (b, 0)),
            pl.BlockSpec((br, cols), lambda b: (b, 0)),
        ],
        out_specs=pl.BlockSpec((br, cols), lambda b: (b, 0)),
        out_shape=jax.ShapeDtypeStruct((rows, cols), jnp.float32),
    )(t_arr, ssum, w, scores)
    return out


# speedup vs baseline: 107.6442x; 2.9238x over previous
import functools

import jax
import jax.numpy as jnp
from jax import lax
from jax.experimental import pallas as pl
from jax.experimental.pallas import tpu as pltpu
from jax.experimental.pallas import tpu_sc as plsc

_PRUNE_RATE = 0.5
_NW = 32
_CHUNK = 8192
_UN = 8
_HI_BITS = 16
_LO_BITS = 15
_HI_BINS = 1 << _HI_BITS
_LO_BINS = 1 << _LO_BITS


def _sc_mesh():
    return plsc.VectorSubcoreMesh(core_axis_name="c", subcore_axis_name="s")


def _make_hist_hi(n):
    per_w = n // _NW
    nch = per_w // _CHUNK

    @functools.partial(
        pl.kernel,
        mesh=_sc_mesh(),
        compiler_params=pltpu.CompilerParams(needs_layout_passes=False),
        out_type=jax.ShapeDtypeStruct((_NW, _HI_BINS), jnp.int32),
        scratch_types=[
            pltpu.VMEM((_HI_BINS,), jnp.int32),
            pltpu.VMEM((_CHUNK,), jnp.float32),
            pltpu.VMEM((_CHUNK,), jnp.float32),
            pltpu.SemaphoreType.DMA,
            pltpu.SemaphoreType.DMA,
        ],
    )
    def hist_kernel(s_hbm, out_hbm, hist_v, buf0, buf1, sem0, sem1):
        wid = lax.axis_index("c") * 16 + lax.axis_index("s")
        base = wid * per_w
        bufs = (buf0, buf1)
        sems = (sem0, sem1)
        zeros16 = jnp.zeros((16,), jnp.int32)
        ones16 = jnp.ones((16,), jnp.int32)

        def zbody(j, c):
            hist_v[pl.ds(j * 16, 16)] = zeros16
            return c

        lax.fori_loop(0, _HI_BINS // 16, zbody, 0)

        def copy(i, b):
            return pltpu.make_async_copy(
                s_hbm.at[pl.ds(base + i * _CHUNK, _CHUNK)], bufs[b], sems[b]
            )

        def process(b):
            buf = bufs[b]

            def body(j, c):
                off = j * (16 * _UN)
                for t in range(_UN):
                    v = buf[pl.ds(off + t * 16, 16)]
                    bits = lax.bitcast_convert_type(v, jnp.int32)
                    u = bits & jnp.int32(0x7FFFFFFF)
                    idv = u >> _LO_BITS
                    plsc.addupdate_scatter(hist_v, [idv], ones16)
                return c

            lax.fori_loop(0, _CHUNK // (16 * _UN), body, 0)

        copy(0, 0).start()
        copy(1, 1).start()

        def outer(ii, c):
            for b in range(2):
                i = ii * 2 + b
                copy(i, b).wait()
                process(b)
                copy(i + 2, b).start()
            return c

        lax.fori_loop(0, nch // 2 - 1, outer, 0)
        for b in range(2):
            copy(nch - 2 + b, b).wait()
            process(b)

        pltpu.sync_copy(hist_v, out_hbm.at[wid])

    return hist_kernel


def _make_hist_lo(n):
    per_w = n // _NW
    nch = per_w // _CHUNK

    @functools.partial(
        pl.kernel,
        mesh=_sc_mesh(),
        compiler_params=pltpu.CompilerParams(needs_layout_passes=False),
        out_type=jax.ShapeDtypeStruct((_NW, _LO_BINS), jnp.int32),
        scratch_types=[
            pltpu.VMEM((_LO_BINS,), jnp.int32),
            pltpu.VMEM((128,), jnp.int32),
            pltpu.VMEM((_CHUNK,), jnp.float32),
            pltpu.VMEM((_CHUNK,), jnp.float32),
            pltpu.SemaphoreType.DMA,
            pltpu.SemaphoreType.DMA,
        ],
    )
    def low_kernel(s_hbm, bf_hbm, out_hbm, hist_v, bvec_v, buf0, buf1, sem0, sem1):
        wid = lax.axis_index("c") * 16 + lax.axis_index("s")
        base = wid * per_w
        bufs = (buf0, buf1)
        sems = (sem0, sem1)
        zeros16 = jnp.zeros((16,), jnp.int32)
        ones16 = jnp.ones((16,), jnp.int32)

        pltpu.sync_copy(bf_hbm.at[0], bvec_v)
        bvec = bvec_v[pl.ds(0, 16)]

        def zbody(j, c):
            hist_v[pl.ds(j * 16, 16)] = zeros16
            return c

        lax.fori_loop(0, _LO_BINS // 16, zbody, 0)

        def copy(i, b):
            return pltpu.make_async_copy(
                s_hbm.at[pl.ds(base + i * _CHUNK, _CHUNK)], bufs[b], sems[b]
            )

        def process(b):
            buf = bufs[b]

            def body(j, c):
                off = j * (16 * _UN)
                for t in range(_UN):
                    v = buf[pl.ds(off + t * 16, 16)]
                    bits = lax.bitcast_convert_type(v, jnp.int32)
                    u = bits & jnp.int32(0x7FFFFFFF)
                    m = (u >> _LO_BITS) == bvec
                    idlo = u & jnp.int32(_LO_BINS - 1)
                    plsc.addupdate_scatter(hist_v, [idlo], ones16, mask=m)
                return c

            lax.fori_loop(0, _CHUNK // (16 * _UN), body, 0)

        copy(0, 0).start()
        copy(1, 1).start()

        def outer(ii, c):
            for b in range(2):
                i = ii * 2 + b
                copy(i, b).wait()
                process(b)
                copy(i + 2, b).start()
            return c

        lax.fori_loop(0, nch // 2 - 1, outer, 0)
        for b in range(2):
            copy(nch - 2 + b, b).wait()
            process(b)

        pltpu.sync_copy(hist_v, out_hbm.at[wid])

    return low_kernel


def _suffix_find(H, need):
    R = H.shape[0]
    rows = jnp.sum(H, axis=1)
    ri = lax.broadcasted_iota(jnp.int32, (R, R), 0)
    rj = lax.broadcasted_iota(jnp.int32, (R, R), 1)
    sxr = jnp.sum(jnp.where(rj > ri, rows[None, :], 0), axis=1)
    rstar = jnp.sum(jnp.where(sxr >= need, jnp.int32(1), jnp.int32(0)))
    sxr_s = jnp.sum(jnp.where(lax.iota(jnp.int32, R) == rstar, sxr, 0))
    need2 = need - sxr_s
    rowio = lax.broadcasted_iota(jnp.int32, (R, 128), 0)
    hrow = jnp.sum(jnp.where(rowio == rstar, H, 0), axis=0)
    li = lax.broadcasted_iota(jnp.int32, (128, 128), 0)
    lj = lax.broadcasted_iota(jnp.int32, (128, 128), 1)
    sxl = jnp.sum(jnp.where(lj > li, hrow[None, :], 0), axis=1)
    lstar = jnp.sum(jnp.where(sxl >= need2, jnp.int32(1), jnp.int32(0)))
    sxl_s = jnp.sum(jnp.where(lax.iota(jnp.int32, 128) == lstar, sxl, 0))
    need3 = need2 - sxl_s
    return rstar * 128 + lstar, need3


def _find_hi_kernel(h_ref, out_ref, *, k):
    H = jnp.sum(h_ref[...], axis=0)
    bstar, need = _suffix_find(H, jnp.int32(k))
    rio = lax.broadcasted_iota(jnp.int32, (8, 128), 0)
    out_ref[...] = jnp.where(rio == 0, bstar, jnp.where(rio == 1, need, 0))


def _find_lo_kernel(h_ref, bf_ref, out_ref):
    H = jnp.sum(h_ref[...], axis=0)
    need = bf_ref[1, 0]
    tlow, _ = _suffix_find(H, need)
    tstar = (bf_ref[0, 0] << _LO_BITS) | tlow
    out_ref[...] = jnp.full((8, 128), tstar, jnp.int32)


def _sum_kernel(t_ref, w_ref, s_ref, out_ref, acc_ref, *, nb):
    b = pl.program_id(0)

    @pl.when(b == 0)
    def _init():
        acc_ref[0] = jnp.float32(0.0)

    bits = lax.bitcast_convert_type(s_ref[...], jnp.int32)
    u = bits & jnp.int32(0x7FFFFFFF)
    t = t_ref[0, 0]
    acc_ref[0] += jnp.sum(jnp.where(u >= t, jnp.abs(w_ref[...]), 0.0))

    @pl.when(b == nb - 1)
    def _emit():
        out_ref[...] = jnp.full_like(out_ref, acc_ref[0])


def _out_kernel(t_ref, ssum_ref, w_ref, s_ref, o_ref, *, inv_k):
    bits = lax.bitcast_convert_type(s_ref[...], jnp.int32)
    u = bits & jnp.int32(0x7FFFFFFF)
    t = t_ref[0, 0]
    alpha = ssum_ref[0, 0] * inv_k
    w = w_ref[...]
    o_ref[...] = jnp.where(u >= t, alpha * jnp.sign(w), jnp.float32(0.0))


def kernel(w, scores):
    rows, cols = scores.shape
    n = rows * cols
    num_unpruned = int(_PRUNE_RATE * n)
    topk = n - int((1.0 - _PRUNE_RATE) * n)
    br = min(256, rows)
    nb = rows // br

    sflat = scores.reshape(-1)

    hist_hi = _make_hist_hi(n)(sflat)
    bf = pl.pallas_call(
        functools.partial(_find_hi_kernel, k=topk),
        grid=(1,),
        in_specs=[pl.BlockSpec((_NW, 512, 128), lambda i: (0, 0, 0))],
        out_specs=pl.BlockSpec((8, 128), lambda i: (0, 0)),
        out_shape=jax.ShapeDtypeStruct((8, 128), jnp.int32),
    )(hist_hi.reshape(_NW, 512, 128))

    hist_lo = _make_hist_lo(n)(sflat, bf)
    t_arr = pl.pallas_call(
        _find_lo_kernel,
        grid=(1,),
        in_specs=[
            pl.BlockSpec((_NW, 256, 128), lambda i: (0, 0, 0)),
            pl.BlockSpec((8, 128), lambda i: (0, 0)),
        ],
        out_specs=pl.BlockSpec((8, 128), lambda i: (0, 0)),
        out_shape=jax.ShapeDtypeStruct((8, 128), jnp.int32),
    )(hist_lo.reshape(_NW, 256, 128), bf)

    ssum = pl.pallas_call(
        functools.partial(_sum_kernel, nb=nb),
        grid=(nb,),
        in_specs=[
            pl.BlockSpec((8, 128), lambda b: (0, 0)),
            pl.BlockSpec((br, cols), lambda b: (b, 0)),
            pl.BlockSpec((br, cols), lambda b: (b, 0)),
        ],
        out_specs=pl.BlockSpec((8, 128), lambda b: (0, 0)),
        out_shape=jax.ShapeDtypeStruct((8, 128), jnp.float32),
        scratch_shapes=[pltpu.SMEM((1,), jnp.float32)],
    )(t_arr, w, scores)

    out = pl.pallas_call(
        functools.partial(_out_kernel, inv_k=1.0 / num_unpruned),
        grid=(nb,),
        in_specs=[
            pl.BlockSpec((8, 128), lambda b: (0, 0)),
            pl.BlockSpec((8, 128), lambda b: (0, 0)),
            pl.BlockSpec((br, cols), lambda b: (b, 0)),
            pl.BlockSpec((br, cols), lambda b: (b, 0)),
        ],
        out_specs=pl.BlockSpec((br, cols), lambda b: (b, 0)),
        out_shape=jax.ShapeDtypeStruct((rows, cols), jnp.float32),
    )(t_arr, ssum, w, scores)
    return out


# speedup vs baseline: 273.7842x; 2.5434x over previous
import functools

import jax
import jax.numpy as jnp
from jax import lax
from jax.experimental import pallas as pl
from jax.experimental.pallas import tpu as pltpu
from jax.experimental.pallas import tpu_sc as plsc

_PRUNE_RATE = 0.5
_NW = 32
_CHUNK = 8192
_UN = 8
_HI_BITS = 16
_LO_BITS = 15
_HI_BINS = 1 << _HI_BITS
_LO_BINS = 1 << _LO_BITS


def _sc_mesh():
    return plsc.VectorSubcoreMesh(core_axis_name="c", subcore_axis_name="s")


def _make_hist_hi(n):
    per_w = n // _NW
    nch = per_w // _CHUNK

    @functools.partial(
        pl.kernel,
        mesh=_sc_mesh(),
        compiler_params=pltpu.CompilerParams(needs_layout_passes=False),
        out_type=jax.ShapeDtypeStruct((_NW, _HI_BINS), jnp.int32),
        scratch_types=[
            pltpu.VMEM((_HI_BINS,), jnp.int32),
            pltpu.VMEM((_CHUNK,), jnp.float32),
            pltpu.VMEM((_CHUNK,), jnp.float32),
            pltpu.SemaphoreType.DMA,
            pltpu.SemaphoreType.DMA,
        ],
    )
    def hist_kernel(s_hbm, out_hbm, hist_v, buf0, buf1, sem0, sem1):
        wid = lax.axis_index("c") * 16 + lax.axis_index("s")
        base = wid * per_w
        bufs = (buf0, buf1)
        sems = (sem0, sem1)
        zeros16 = jnp.zeros((16,), jnp.int32)
        ones16 = jnp.ones((16,), jnp.int32)

        @plsc.parallel_loop(0, _HI_BINS, step=16, unroll=8)
        def _zero(j):
            hist_v[pl.ds(j, 16)] = zeros16

        def copy(i, b):
            return pltpu.make_async_copy(
                s_hbm.at[pl.ds(base + i * _CHUNK, _CHUNK)], bufs[b], sems[b]
            )

        def process(b):
            buf = bufs[b]

            @plsc.parallel_loop(0, _CHUNK, step=16, unroll=_UN)
            def _body(i):
                v = buf[pl.ds(i, 16)]
                bits = lax.bitcast_convert_type(v, jnp.int32)
                idv = (bits >> _LO_BITS) & jnp.int32(_HI_BINS - 1)
                plsc.addupdate_scatter(hist_v, [idv], ones16)

        copy(0, 0).start()
        copy(1, 1).start()

        def outer(ii, c):
            for b in range(2):
                i = ii * 2 + b
                copy(i, b).wait()
                process(b)
                copy(i + 2, b).start()
            return c

        lax.fori_loop(0, nch // 2 - 1, outer, 0)
        for b in range(2):
            copy(nch - 2 + b, b).wait()
            process(b)

        pltpu.sync_copy(hist_v, out_hbm.at[wid])

    return hist_kernel


def _make_hist_lo(n):
    per_w = n // _NW
    nch = per_w // _CHUNK

    @functools.partial(
        pl.kernel,
        mesh=_sc_mesh(),
        compiler_params=pltpu.CompilerParams(needs_layout_passes=False),
        out_type=jax.ShapeDtypeStruct((_NW, _LO_BINS), jnp.int32),
        scratch_types=[
            pltpu.VMEM((_LO_BINS,), jnp.int32),
            pltpu.VMEM((128,), jnp.int32),
            pltpu.VMEM((_CHUNK,), jnp.float32),
            pltpu.VMEM((_CHUNK,), jnp.float32),
            pltpu.SemaphoreType.DMA,
            pltpu.SemaphoreType.DMA,
        ],
    )
    def low_kernel(s_hbm, bf_hbm, out_hbm, hist_v, bvec_v, buf0, buf1, sem0, sem1):
        wid = lax.axis_index("c") * 16 + lax.axis_index("s")
        base = wid * per_w
        bufs = (buf0, buf1)
        sems = (sem0, sem1)
        zeros16 = jnp.zeros((16,), jnp.int32)
        ones16 = jnp.ones((16,), jnp.int32)

        pltpu.sync_copy(bf_hbm.at[0], bvec_v)
        bvec = bvec_v[pl.ds(0, 16)]

        @plsc.parallel_loop(0, _LO_BINS, step=16, unroll=8)
        def _zero(j):
            hist_v[pl.ds(j, 16)] = zeros16

        def copy(i, b):
            return pltpu.make_async_copy(
                s_hbm.at[pl.ds(base + i * _CHUNK, _CHUNK)], bufs[b], sems[b]
            )

        def process(b):
            buf = bufs[b]

            @plsc.parallel_loop(0, _CHUNK, step=16, unroll=_UN)
            def _body(i):
                v = buf[pl.ds(i, 16)]
                bits = lax.bitcast_convert_type(v, jnp.int32)
                idhi = (bits >> _LO_BITS) & jnp.int32(_HI_BINS - 1)
                m = idhi == bvec
                idlo = bits & jnp.int32(_LO_BINS - 1)
                plsc.addupdate_scatter(hist_v, [idlo], ones16, mask=m)

        copy(0, 0).start()
        copy(1, 1).start()

        def outer(ii, c):
            for b in range(2):
                i = ii * 2 + b
                copy(i, b).wait()
                process(b)
                copy(i + 2, b).start()
            return c

        lax.fori_loop(0, nch // 2 - 1, outer, 0)
        for b in range(2):
            copy(nch - 2 + b, b).wait()
            process(b)

        pltpu.sync_copy(hist_v, out_hbm.at[wid])

    return low_kernel


def _suffix_find(H, need):
    R = H.shape[0]
    rows = jnp.sum(H, axis=1)
    ri = lax.broadcasted_iota(jnp.int32, (R, R), 0)
    rj = lax.broadcasted_iota(jnp.int32, (R, R), 1)
    sxr = jnp.sum(jnp.where(rj > ri, rows[None, :], 0), axis=1)
    rstar = jnp.sum(jnp.where(sxr >= need, jnp.int32(1), jnp.int32(0)))
    sxr_s = jnp.sum(jnp.where(lax.iota(jnp.int32, R) == rstar, sxr, 0))
    need2 = need - sxr_s
    rowio = lax.broadcasted_iota(jnp.int32, (R, 128), 0)
    hrow = jnp.sum(jnp.where(rowio == rstar, H, 0), axis=0)
    li = lax.broadcasted_iota(jnp.int32, (128, 128), 0)
    lj = lax.broadcasted_iota(jnp.int32, (128, 128), 1)
    sxl = jnp.sum(jnp.where(lj > li, hrow[None, :], 0), axis=1)
    lstar = jnp.sum(jnp.where(sxl >= need2, jnp.int32(1), jnp.int32(0)))
    sxl_s = jnp.sum(jnp.where(lax.iota(jnp.int32, 128) == lstar, sxl, 0))
    need3 = need2 - sxl_s
    return rstar * 128 + lstar, need3


def _find_hi_kernel(h_ref, out_ref, *, k):
    H = jnp.sum(h_ref[...], axis=0)
    bstar, need = _suffix_find(H, jnp.int32(k))
    rio = lax.broadcasted_iota(jnp.int32, (8, 128), 0)
    out_ref[...] = jnp.where(rio == 0, bstar, jnp.where(rio == 1, need, 0))


def _find_lo_kernel(h_ref, bf_ref, out_ref):
    H = jnp.sum(h_ref[...], axis=0)
    need = bf_ref[1, 0]
    tlow, _ = _suffix_find(H, need)
    tstar = (bf_ref[0, 0] << _LO_BITS) | tlow
    out_ref[...] = jnp.full((8, 128), tstar, jnp.int32)


def _sum_kernel(t_ref, w_ref, s_ref, out_ref, acc_ref, *, nb):
    b = pl.program_id(0)

    @pl.when(b == 0)
    def _init():
        acc_ref[0] = jnp.float32(0.0)

    bits = lax.bitcast_convert_type(s_ref[...], jnp.int32)
    u = bits & jnp.int32(0x7FFFFFFF)
    t = t_ref[0, 0]
    acc_ref[0] += jnp.sum(jnp.where(u >= t, jnp.abs(w_ref[...]), 0.0))

    @pl.when(b == nb - 1)
    def _emit():
        out_ref[...] = jnp.full_like(out_ref, acc_ref[0])


def _out_kernel(t_ref, ssum_ref, w_ref, s_ref, o_ref, *, inv_k):
    bits = lax.bitcast_convert_type(s_ref[...], jnp.int32)
    u = bits & jnp.int32(0x7FFFFFFF)
    t = t_ref[0, 0]
    alpha = ssum_ref[0, 0] * inv_k
    w = w_ref[...]
    o_ref[...] = jnp.where(u >= t, alpha * jnp.sign(w), jnp.float32(0.0))


def kernel(w, scores):
    rows, cols = scores.shape
    n = rows * cols
    num_unpruned = int(_PRUNE_RATE * n)
    topk = n - int((1.0 - _PRUNE_RATE) * n)
    br = min(256, rows)
    nb = rows // br

    sflat = scores.reshape(-1)

    hist_hi = _make_hist_hi(n)(sflat)
    bf = pl.pallas_call(
        functools.partial(_find_hi_kernel, k=topk),
        grid=(1,),
        in_specs=[pl.BlockSpec((_NW, 512, 128), lambda i: (0, 0, 0))],
        out_specs=pl.BlockSpec((8, 128), lambda i: (0, 0)),
        out_shape=jax.ShapeDtypeStruct((8, 128), jnp.int32),
    )(hist_hi.reshape(_NW, 512, 128))

    hist_lo = _make_hist_lo(n)(sflat, bf)
    t_arr = pl.pallas_call(
        _find_lo_kernel,
        grid=(1,),
        in_specs=[
            pl.BlockSpec((_NW, 256, 128), lambda i: (0, 0, 0)),
            pl.BlockSpec((8, 128), lambda i: (0, 0)),
        ],
        out_specs=pl.BlockSpec((8, 128), lambda i: (0, 0)),
        out_shape=jax.ShapeDtypeStruct((8, 128), jnp.int32),
    )(hist_lo.reshape(_NW, 256, 128), bf)

    ssum = pl.pallas_call(
        functools.partial(_sum_kernel, nb=nb),
        grid=(nb,),
        in_specs=[
            pl.BlockSpec((8, 128), lambda b: (0, 0)),
            pl.BlockSpec((br, cols), lambda b: (b, 0)),
            pl.BlockSpec((br, cols), lambda b: (b, 0)),
        ],
        out_specs=pl.BlockSpec((8, 128), lambda b: (0, 0)),
        out_shape=jax.ShapeDtypeStruct((8, 128), jnp.float32),
        scratch_shapes=[pltpu.SMEM((1,), jnp.float32)],
    )(t_arr, w, scores)

    out = pl.pallas_call(
        functools.partial(_out_kernel, inv_k=1.0 / num_unpruned),
        grid=(nb,),
        in_specs=[
            pl.BlockSpec((8, 128), lambda b: (0, 0)),
            pl.BlockSpec((8, 128), lambda b: (0, 0)),
            pl.BlockSpec((br, cols), lambda b: (b, 0)),
            pl.BlockSpec((br, cols), lambda b: (b, 0)),
        ],
        out_specs=pl.BlockSpec((br, cols), lambda b: (b, 0)),
        out_shape=jax.ShapeDtypeStruct((rows, cols), jnp.float32),
    )(t_arr, ssum, w, scores)
    return out


# speedup vs baseline: 352.9814x; 1.2893x over previous
import functools

import jax
import jax.numpy as jnp
from jax import lax
from jax.experimental import pallas as pl
from jax.experimental.pallas import tpu as pltpu
from jax.experimental.pallas import tpu_sc as plsc

_PRUNE_RATE = 0.5
_NW = 32
_CROWS = 8
_HI_BITS = 16
_LO_BITS = 15
_HI_BINS = 1 << _HI_BITS
_LO_BINS = 1 << _LO_BITS


def _sc_mesh():
    return plsc.VectorSubcoreMesh(core_axis_name="c", subcore_axis_name="s")


def _make_hist_hi(rows, cols):
    ccols = min(2048, cols)
    nch_total = (rows // _CROWS) * (cols // ccols)
    nch = nch_total // _NW
    cpr = cols // ccols

    @functools.partial(
        pl.kernel,
        mesh=_sc_mesh(),
        compiler_params=pltpu.CompilerParams(needs_layout_passes=False),
        out_type=jax.ShapeDtypeStruct((_NW, _HI_BINS), jnp.int32),
        scratch_types=[
            pltpu.VMEM((_HI_BINS,), jnp.int32),
            pltpu.VMEM((_CROWS, 2048), jnp.float32),
            pltpu.VMEM((_CROWS, 2048), jnp.float32),
            pltpu.SemaphoreType.DMA,
            pltpu.SemaphoreType.DMA,
        ],
    )
    def hist_kernel(s_hbm, out_hbm, hist_v, buf0, buf1, sem0, sem1):
        wid = lax.axis_index("c") * 16 + lax.axis_index("s")
        base_g = wid * nch
        bufs = (buf0, buf1)
        sems = (sem0, sem1)
        zeros16 = jnp.zeros((16,), jnp.int32)
        ones16 = jnp.ones((16,), jnp.int32)

        @plsc.parallel_loop(0, _HI_BINS, step=16, unroll=8)
        def _zero(j):
            hist_v[pl.ds(j, 16)] = zeros16

        def copy(i, b):
            g = base_g + i
            rb = g // cpr
            h = g % cpr
            return pltpu.make_async_copy(
                s_hbm.at[pl.ds(rb * _CROWS, _CROWS), pl.ds(h * ccols, ccols)],
                bufs[b],
                sems[b],
            )

        def process(b):
            buf = bufs[b]

            @plsc.parallel_loop(0, ccols, step=16, unroll=2)
            def _body(i):
                for r in range(_CROWS):
                    v = buf[r, pl.ds(i, 16)]
                    bits = lax.bitcast_convert_type(v, jnp.int32)
                    idv = (bits >> _LO_BITS) & jnp.int32(_HI_BINS - 1)
                    plsc.addupdate_scatter(hist_v, [idv], ones16)

        copy(0, 0).start()
        copy(1, 1).start()

        def outer(ii, c):
            for b in range(2):
                i = ii * 2 + b
                copy(i, b).wait()
                process(b)
                copy(i + 2, b).start()
            return c

        lax.fori_loop(0, nch // 2 - 1, outer, 0)
        for b in range(2):
            copy(nch - 2 + b, b).wait()
            process(b)

        pltpu.sync_copy(hist_v, out_hbm.at[wid])

    return hist_kernel


def _make_hist_lo(rows, cols):
    ccols = min(2048, cols)
    nch_total = (rows // _CROWS) * (cols // ccols)
    nch = nch_total // _NW
    cpr = cols // ccols

    @functools.partial(
        pl.kernel,
        mesh=_sc_mesh(),
        compiler_params=pltpu.CompilerParams(needs_layout_passes=False),
        out_type=jax.ShapeDtypeStruct((_NW, _LO_BINS), jnp.int32),
        scratch_types=[
            pltpu.VMEM((_LO_BINS,), jnp.int32),
            pltpu.VMEM((128,), jnp.int32),
            pltpu.VMEM((_CROWS, 2048), jnp.float32),
            pltpu.VMEM((_CROWS, 2048), jnp.float32),
            pltpu.SemaphoreType.DMA,
            pltpu.SemaphoreType.DMA,
        ],
    )
    def low_kernel(s_hbm, bf_hbm, out_hbm, hist_v, bvec_v, buf0, buf1, sem0, sem1):
        wid = lax.axis_index("c") * 16 + lax.axis_index("s")
        base_g = wid * nch
        bufs = (buf0, buf1)
        sems = (sem0, sem1)
        zeros16 = jnp.zeros((16,), jnp.int32)
        ones16 = jnp.ones((16,), jnp.int32)

        pltpu.sync_copy(bf_hbm.at[0], bvec_v)
        bvec = bvec_v[pl.ds(0, 16)]

        @plsc.parallel_loop(0, _LO_BINS, step=16, unroll=8)
        def _zero(j):
            hist_v[pl.ds(j, 16)] = zeros16

        def copy(i, b):
            g = base_g + i
            rb = g // cpr
            h = g % cpr
            return pltpu.make_async_copy(
                s_hbm.at[pl.ds(rb * _CROWS, _CROWS), pl.ds(h * ccols, ccols)],
                bufs[b],
                sems[b],
            )

        def process(b):
            buf = bufs[b]

            @plsc.parallel_loop(0, ccols, step=16, unroll=2)
            def _body(i):
                for r in range(_CROWS):
                    v = buf[r, pl.ds(i, 16)]
                    bits = lax.bitcast_convert_type(v, jnp.int32)
                    idhi = (bits >> _LO_BITS) & jnp.int32(_HI_BINS - 1)
                    m = idhi == bvec
                    idlo = bits & jnp.int32(_LO_BINS - 1)
                    plsc.addupdate_scatter(hist_v, [idlo], ones16, mask=m)

        copy(0, 0).start()
        copy(1, 1).start()

        def outer(ii, c):
            for b in range(2):
                i = ii * 2 + b
                copy(i, b).wait()
                process(b)
                copy(i + 2, b).start()
            return c

        lax.fori_loop(0, nch // 2 - 1, outer, 0)
        for b in range(2):
            copy(nch - 2 + b, b).wait()
            process(b)

        pltpu.sync_copy(hist_v, out_hbm.at[wid])

    return low_kernel


def _suffix_find(H, need):
    R = H.shape[0]
    rows = jnp.sum(H, axis=1)
    ri = lax.broadcasted_iota(jnp.int32, (R, R), 0)
    rj = lax.broadcasted_iota(jnp.int32, (R, R), 1)
    sxr = jnp.sum(jnp.where(rj > ri, rows[None, :], 0), axis=1)
    rstar = jnp.sum(jnp.where(sxr >= need, jnp.int32(1), jnp.int32(0)))
    sxr_s = jnp.sum(jnp.where(lax.iota(jnp.int32, R) == rstar, sxr, 0))
    need2 = need - sxr_s
    rowio = lax.broadcasted_iota(jnp.int32, (R, 128), 0)
    hrow = jnp.sum(jnp.where(rowio == rstar, H, 0), axis=0)
    li = lax.broadcasted_iota(jnp.int32, (128, 128), 0)
    lj = lax.broadcasted_iota(jnp.int32, (128, 128), 1)
    sxl = jnp.sum(jnp.where(lj > li, hrow[None, :], 0), axis=1)
    lstar = jnp.sum(jnp.where(sxl >= need2, jnp.int32(1), jnp.int32(0)))
    sxl_s = jnp.sum(jnp.where(lax.iota(jnp.int32, 128) == lstar, sxl, 0))
    need3 = need2 - sxl_s
    return rstar * 128 + lstar, need3


def _find_hi_kernel(h_ref, out_ref, *, k):
    H = jnp.sum(h_ref[...], axis=0)
    bstar, need = _suffix_find(H, jnp.int32(k))
    rio = lax.broadcasted_iota(jnp.int32, (8, 128), 0)
    out_ref[...] = jnp.where(rio == 0, bstar, jnp.where(rio == 1, need, 0))


def _find_lo_kernel(h_ref, bf_ref, out_ref):
    H = jnp.sum(h_ref[...], axis=0)
    need = bf_ref[1, 0]
    tlow, _ = _suffix_find(H, need)
    tstar = (bf_ref[0, 0] << _LO_BITS) | tlow
    out_ref[...] = jnp.full((8, 128), tstar, jnp.int32)


def _sum_kernel(t_ref, w_ref, s_ref, out_ref, acc_ref, *, nb):
    b = pl.program_id(0)

    @pl.when(b == 0)
    def _init():
        acc_ref[0] = jnp.float32(0.0)

    bits = lax.bitcast_convert_type(s_ref[...], jnp.int32)
    u = bits & jnp.int32(0x7FFFFFFF)
    t = t_ref[0, 0]
    acc_ref[0] += jnp.sum(jnp.where(u >= t, jnp.abs(w_ref[...]), 0.0))

    @pl.when(b == nb - 1)
    def _emit():
        out_ref[...] = jnp.full_like(out_ref, acc_ref[0])


def _out_kernel(t_ref, ssum_ref, w_ref, s_ref, o_ref, *, inv_k):
    bits = lax.bitcast_convert_type(s_ref[...], jnp.int32)
    u = bits & jnp.int32(0x7FFFFFFF)
    t = t_ref[0, 0]
    alpha = ssum_ref[0, 0] * inv_k
    w = w_ref[...]
    o_ref[...] = jnp.where(u >= t, alpha * jnp.sign(w), jnp.float32(0.0))


def kernel(w, scores):
    rows, cols = scores.shape
    n = rows * cols
    num_unpruned = int(_PRUNE_RATE * n)
    topk = n - int((1.0 - _PRUNE_RATE) * n)
    br = min(256, rows)
    nb = rows // br

    hist_hi = _make_hist_hi(rows, cols)(scores)
    bf = pl.pallas_call(
        functools.partial(_find_hi_kernel, k=topk),
        grid=(1,),
        in_specs=[pl.BlockSpec((_NW, 512, 128), lambda i: (0, 0, 0))],
        out_specs=pl.BlockSpec((8, 128), lambda i: (0, 0)),
        out_shape=jax.ShapeDtypeStruct((8, 128), jnp.int32),
    )(hist_hi.reshape(_NW, 512, 128))

    hist_lo = _make_hist_lo(rows, cols)(scores, bf)
    t_arr = pl.pallas_call(
        _find_lo_kernel,
        grid=(1,),
        in_specs=[
            pl.BlockSpec((_NW, 256, 128), lambda i: (0, 0, 0)),
            pl.BlockSpec((8, 128), lambda i: (0, 0)),
        ],
        out_specs=pl.BlockSpec((8, 128), lambda i: (0, 0)),
        out_shape=jax.ShapeDtypeStruct((8, 128), jnp.int32),
    )(hist_lo.reshape(_NW, 256, 128), bf)

    ssum = pl.pallas_call(
        functools.partial(_sum_kernel, nb=nb),
        grid=(nb,),
        in_specs=[
            pl.BlockSpec((8, 128), lambda b: (0, 0)),
            pl.BlockSpec((br, cols), lambda b: (b, 0)),
            pl.BlockSpec((br, cols), lambda b: (b, 0)),
        ],
        out_specs=pl.BlockSpec((8, 128), lambda b: (0, 0)),
        out_shape=jax.ShapeDtypeStruct((8, 128), jnp.float32),
        scratch_shapes=[pltpu.SMEM((1,), jnp.float32)],
    )(t_arr, w, scores)

    out = pl.pallas_call(
        functools.partial(_out_kernel, inv_k=1.0 / num_unpruned),
        grid=(nb,),
        in_specs=[
            pl.BlockSpec((8, 128), lambda b: (0, 0)),
            pl.BlockSpec((8, 128), lambda b: (0, 0)),
            pl.BlockSpec((br, cols), lambda b: (b, 0)),
            pl.BlockSpec((br, cols), lambda b: (b, 0)),
        ],
        out_specs=pl.BlockSpec((br, cols), lambda b: (b, 0)),
        out_shape=jax.ShapeDtypeStruct((rows, cols), jnp.float32),
    )(t_arr, ssum, w, scores)
    return out


# speedup vs baseline: 354.6743x; 1.0048x over previous
import functools

import jax
import jax.numpy as jnp
from jax import lax
from jax.experimental import pallas as pl
from jax.experimental.pallas import tpu as pltpu
from jax.experimental.pallas import tpu_sc as plsc

_PRUNE_RATE = 0.5
_NW = 32
_CROWS = 8
_HI_BITS = 16
_LO_BITS = 15
_HI_BINS = 1 << _HI_BITS
_LO_BINS = 1 << _LO_BITS


def _sc_mesh():
    return plsc.VectorSubcoreMesh(core_axis_name="c", subcore_axis_name="s")


def _make_hist_hi(rows, cols):
    ccols = min(2048, cols)
    nch_total = (rows // _CROWS) * (cols // ccols)
    nch = nch_total // _NW
    cpr = cols // ccols

    @functools.partial(
        pl.kernel,
        mesh=_sc_mesh(),
        compiler_params=pltpu.CompilerParams(needs_layout_passes=False),
        out_type=jax.ShapeDtypeStruct((_NW, _HI_BINS), jnp.int32),
        scratch_types=[
            pltpu.VMEM((_HI_BINS,), jnp.int32),
            pltpu.VMEM((_CROWS, 2048), jnp.float32),
            pltpu.VMEM((_CROWS, 2048), jnp.float32),
            pltpu.SemaphoreType.DMA,
            pltpu.SemaphoreType.DMA,
        ],
    )
    def hist_kernel(s_hbm, out_hbm, hist_v, buf0, buf1, sem0, sem1):
        wid = lax.axis_index("c") * 16 + lax.axis_index("s")
        base_g = wid * nch
        bufs = (buf0, buf1)
        sems = (sem0, sem1)
        zeros16 = jnp.zeros((16,), jnp.int32)
        ones16 = jnp.ones((16,), jnp.int32)

        @plsc.parallel_loop(0, _HI_BINS, step=16, unroll=8)
        def _zero(j):
            hist_v[pl.ds(j, 16)] = zeros16

        def copy(i, b):
            g = base_g + i
            rb = g // cpr
            h = g % cpr
            return pltpu.make_async_copy(
                s_hbm.at[pl.ds(rb * _CROWS, _CROWS), pl.ds(h * ccols, ccols)],
                bufs[b],
                sems[b],
            )

        def process(b):
            buf = bufs[b]

            @plsc.parallel_loop(0, ccols, step=16, unroll=4)
            def _body(i):
                for r in range(_CROWS):
                    v = buf[r, pl.ds(i, 16)]
                    bits = lax.bitcast_convert_type(v, jnp.int32)
                    idv = (bits >> _LO_BITS) & jnp.int32(_HI_BINS - 1)
                    plsc.addupdate_scatter(hist_v, [idv], ones16)

        copy(0, 0).start()
        copy(1, 1).start()

        def outer(ii, c):
            for b in range(2):
                i = ii * 2 + b
                copy(i, b).wait()
                process(b)
                copy(i + 2, b).start()
            return c

        lax.fori_loop(0, nch // 2 - 1, outer, 0)
        for b in range(2):
            copy(nch - 2 + b, b).wait()
            process(b)

        pltpu.sync_copy(hist_v, out_hbm.at[wid])

    return hist_kernel


def _make_hist_lo(rows, cols):
    ccols = min(2048, cols)
    nch_total = (rows // _CROWS) * (cols // ccols)
    nch = nch_total // _NW
    cpr = cols // ccols

    @functools.partial(
        pl.kernel,
        mesh=_sc_mesh(),
        compiler_params=pltpu.CompilerParams(needs_layout_passes=False),
        out_type=jax.ShapeDtypeStruct((_NW, _LO_BINS), jnp.int32),
        scratch_types=[
            pltpu.VMEM((_LO_BINS,), jnp.int32),
            pltpu.VMEM((128,), jnp.int32),
            pltpu.VMEM((_CROWS, 2048), jnp.float32),
            pltpu.VMEM((_CROWS, 2048), jnp.float32),
            pltpu.SemaphoreType.DMA,
            pltpu.SemaphoreType.DMA,
        ],
    )
    def low_kernel(s_hbm, bf_hbm, out_hbm, hist_v, bvec_v, buf0, buf1, sem0, sem1):
        wid = lax.axis_index("c") * 16 + lax.axis_index("s")
        base_g = wid * nch
        bufs = (buf0, buf1)
        sems = (sem0, sem1)
        zeros16 = jnp.zeros((16,), jnp.int32)
        ones16 = jnp.ones((16,), jnp.int32)

        pltpu.sync_copy(bf_hbm.at[0], bvec_v)
        bvec = bvec_v[pl.ds(0, 16)]

        @plsc.parallel_loop(0, _LO_BINS, step=16, unroll=8)
        def _zero(j):
            hist_v[pl.ds(j, 16)] = zeros16

        def copy(i, b):
            g = base_g + i
            rb = g // cpr
            h = g % cpr
            return pltpu.make_async_copy(
                s_hbm.at[pl.ds(rb * _CROWS, _CROWS), pl.ds(h * ccols, ccols)],
                bufs[b],
                sems[b],
            )

        def process(b):
            buf = bufs[b]

            @plsc.parallel_loop(0, ccols, step=16, unroll=4)
            def _body(i):
                for r in range(_CROWS):
                    v = buf[r, pl.ds(i, 16)]
                    bits = lax.bitcast_convert_type(v, jnp.int32)
                    idhi = (bits >> _LO_BITS) & jnp.int32(_HI_BINS - 1)
                    m = idhi == bvec
                    idlo = bits & jnp.int32(_LO_BINS - 1)
                    plsc.addupdate_scatter(hist_v, [idlo], ones16, mask=m)

        copy(0, 0).start()
        copy(1, 1).start()

        def outer(ii, c):
            for b in range(2):
                i = ii * 2 + b
                copy(i, b).wait()
                process(b)
                copy(i + 2, b).start()
            return c

        lax.fori_loop(0, nch // 2 - 1, outer, 0)
        for b in range(2):
            copy(nch - 2 + b, b).wait()
            process(b)

        pltpu.sync_copy(hist_v, out_hbm.at[wid])

    return low_kernel


def _suffix_find(H, need):
    R = H.shape[0]
    rows = jnp.sum(H, axis=1)
    ri = lax.broadcasted_iota(jnp.int32, (R, R), 0)
    rj = lax.broadcasted_iota(jnp.int32, (R, R), 1)
    sxr = jnp.sum(jnp.where(rj > ri, rows[None, :], 0), axis=1)
    rstar = jnp.sum(jnp.where(sxr >= need, jnp.int32(1), jnp.int32(0)))
    sxr_s = jnp.sum(jnp.where(lax.iota(jnp.int32, R) == rstar, sxr, 0))
    need2 = need - sxr_s
    rowio = lax.broadcasted_iota(jnp.int32, (R, 128), 0)
    hrow = jnp.sum(jnp.where(rowio == rstar, H, 0), axis=0)
    li = lax.broadcasted_iota(jnp.int32, (128, 128), 0)
    lj = lax.broadcasted_iota(jnp.int32, (128, 128), 1)
    sxl = jnp.sum(jnp.where(lj > li, hrow[None, :], 0), axis=1)
    lstar = jnp.sum(jnp.where(sxl >= need2, jnp.int32(1), jnp.int32(0)))
    sxl_s = jnp.sum(jnp.where(lax.iota(jnp.int32, 128) == lstar, sxl, 0))
    need3 = need2 - sxl_s
    return rstar * 128 + lstar, need3


def _find_hi_kernel(h_ref, out_ref, *, k):
    H = jnp.sum(h_ref[...], axis=0)
    bstar, need = _suffix_find(H, jnp.int32(k))
    rio = lax.broadcasted_iota(jnp.int32, (8, 128), 0)
    out_ref[...] = jnp.where(rio == 0, bstar, jnp.where(rio == 1, need, 0))


def _find_lo_kernel(h_ref, bf_ref, out_ref):
    H = jnp.sum(h_ref[...], axis=0)
    need = bf_ref[1, 0]
    tlow, _ = _suffix_find(H, need)
    tstar = (bf_ref[0, 0] << _LO_BITS) | tlow
    out_ref[...] = jnp.full((8, 128), tstar, jnp.int32)


def _sum_kernel(t_ref, w_ref, s_ref, out_ref, acc_ref, *, nb):
    b = pl.program_id(0)

    @pl.when(b == 0)
    def _init():
        acc_ref[0] = jnp.float32(0.0)

    bits = lax.bitcast_convert_type(s_ref[...], jnp.int32)
    u = bits & jnp.int32(0x7FFFFFFF)
    t = t_ref[0, 0]
    acc_ref[0] += jnp.sum(jnp.where(u >= t, jnp.abs(w_ref[...]), 0.0))

    @pl.when(b == nb - 1)
    def _emit():
        out_ref[...] = jnp.full_like(out_ref, acc_ref[0])


def _out_kernel(t_ref, ssum_ref, w_ref, s_ref, o_ref, *, inv_k):
    bits = lax.bitcast_convert_type(s_ref[...], jnp.int32)
    u = bits & jnp.int32(0x7FFFFFFF)
    t = t_ref[0, 0]
    alpha = ssum_ref[0, 0] * inv_k
    w = w_ref[...]
    o_ref[...] = jnp.where(u >= t, alpha * jnp.sign(w), jnp.float32(0.0))


def kernel(w, scores):
    rows, cols = scores.shape
    n = rows * cols
    num_unpruned = int(_PRUNE_RATE * n)
    topk = n - int((1.0 - _PRUNE_RATE) * n)
    br = min(256, rows)
    nb = rows // br

    hist_hi = _make_hist_hi(rows, cols)(scores)
    bf = pl.pallas_call(
        functools.partial(_find_hi_kernel, k=topk),
        grid=(1,),
        in_specs=[pl.BlockSpec((_NW, 512, 128), lambda i: (0, 0, 0))],
        out_specs=pl.BlockSpec((8, 128), lambda i: (0, 0)),
        out_shape=jax.ShapeDtypeStruct((8, 128), jnp.int32),
    )(hist_hi.reshape(_NW, 512, 128))

    hist_lo = _make_hist_lo(rows, cols)(scores, bf)
    t_arr = pl.pallas_call(
        _find_lo_kernel,
        grid=(1,),
        in_specs=[
            pl.BlockSpec((_NW, 256, 128), lambda i: (0, 0, 0)),
            pl.BlockSpec((8, 128), lambda i: (0, 0)),
        ],
        out_specs=pl.BlockSpec((8, 128), lambda i: (0, 0)),
        out_shape=jax.ShapeDtypeStruct((8, 128), jnp.int32),
    )(hist_lo.reshape(_NW, 256, 128), bf)

    ssum = pl.pallas_call(
        functools.partial(_sum_kernel, nb=nb),
        grid=(nb,),
        in_specs=[
            pl.BlockSpec((8, 128), lambda b: (0, 0)),
            pl.BlockSpec((br, cols), lambda b: (b, 0)),
            pl.BlockSpec((br, cols), lambda b: (b, 0)),
        ],
        out_specs=pl.BlockSpec((8, 128), lambda b: (0, 0)),
        out_shape=jax.ShapeDtypeStruct((8, 128), jnp.float32),
        scratch_shapes=[pltpu.SMEM((1,), jnp.float32)],
    )(t_arr, w, scores)

    out = pl.pallas_call(
        functools.partial(_out_kernel, inv_k=1.0 / num_unpruned),
        grid=(nb,),
        in_specs=[
            pl.BlockSpec((8, 128), lambda b: (0, 0)),
            pl.BlockSpec((8, 128), lambda b: (0, 0)),
            pl.BlockSpec((br, cols), lambda b: (b, 0)),
            pl.BlockSpec((br, cols), lambda b: (b, 0)),
        ],
        out_specs=pl.BlockSpec((br, cols), lambda b: (b, 0)),
        out_shape=jax.ShapeDtypeStruct((rows, cols), jnp.float32),
    )(t_arr, ssum, w, scores)
    return out
